# baseline (speedup 1.0000x reference)
import functools

import jax
import jax.numpy as jnp
import numpy as np
from jax import lax
from jax.experimental import pallas as pl
from jax.experimental.pallas import tpu as pltpu
from jax.experimental.pallas import tpu_sc as plsc

L = 16
UNROLL = 8
STEP = L * UNROLL


def _cumsum16(scratch, v, iota):
    for k in (1, 2, 4, 8):
        scratch[pl.ds(0, L)] = v
        idx = jnp.maximum(iota - k, 0)
        sh = plsc.load_gather(scratch, [idx])
        v = v + jnp.where(iota >= k, sh, 0)
    return v


def _fetch(ref, i):
    return ref[pl.ds(i, L)][0]


def _search_right(ref, x, log2n):
    def body(_, lohi):
        lo, hi = lohi
        mid = (lo + hi) // 2
        pred = _fetch(ref, mid) <= x
        return jnp.where(pred, mid + 1, lo), jnp.where(pred, hi, mid)

    lo, _ = lax.fori_loop(
        0, log2n, body, (jnp.int32(0), jnp.int32(2 ** log2n)))
    return lo


@functools.lru_cache(maxsize=None)
def _build(B, tgt_len):
    out_size = B * (B - 1) // 2
    n_chunks = 128
    assert out_size % n_chunks == 0
    chunk = out_size // n_chunks
    assert chunk % 8 == 0 and tgt_len == out_size + B
    log2b = B.bit_length() - 1
    assert 2 ** log2b == B
    nvec = B // L

    mesh = plsc.VectorSubcoreMesh(core_axis_name="c", subcore_axis_name="s")
    n_cores = 2
    n_workers = 32
    cpw = n_chunks // n_workers

    fil_s = np.arange(B, dtype=np.int64)
    cs_s = np.cumsum(fil_s)
    ns_s = cs_s - fil_s
    segcnt = []
    for c in range(n_chunks):
        o0_s = c * chunk
        s_lo_s = int(np.searchsorted(cs_s, o0_s, side="right"))
        s_hi_s = int(np.searchsorted(ns_s, o0_s + chunk - 1, side="right"))
        segcnt.append(s_hi_s - s_lo_s)
    bins = [[segcnt[w], [w]] for w in range(n_workers)]
    for c in sorted(range(n_workers, n_chunks), key=lambda c: -segcnt[c]):
        b = min((b for b in bins if len(b[1]) < cpw), key=lambda b: b[0])
        b[0] += segcnt[c]
        b[1].append(c)
    table_np = np.array([b[1] for b in bins], dtype=np.int32).reshape(-1)

    @functools.partial(
        pl.kernel,
        out_type=jax.ShapeDtypeStruct((out_size,), jnp.float32),
        mesh=mesh,
        compiler_params=pltpu.CompilerParams(needs_layout_passes=False),
        scratch_types=[
            pltpu.VMEM((B,), jnp.int32),
            pltpu.VMEM((B,), jnp.int32),
            pltpu.VMEM((B + L,), jnp.int32),
            pltpu.VMEM((B + L,), jnp.int32),
            pltpu.VMEM((B + L,), jnp.int32),
            pltpu.VMEM((L,), jnp.int32),
            pltpu.VMEM((n_chunks + L,), jnp.int32),
            pltpu.VMEM((chunk + B + STEP,), jnp.float32),
            pltpu.VMEM((chunk + B + STEP,), jnp.float32),
            pltpu.VMEM((chunk + STEP,), jnp.float32),
            pltpu.VMEM((chunk + STEP,), jnp.float32),
            pltpu.SemaphoreType.DMA,
            pltpu.SemaphoreType.DMA,
            pltpu.SemaphoreType.DMA,
            pltpu.SemaphoreType.DMA,
        ],
    )
    def ragged_copy(tgt_hbm, acc_hbm, fil_hbm, tbl_hbm, out_hbm,
                    acc_v, fil_v, cs_v, ns_v, sh_v, tmp_v, tbl_v,
                    src_a, src_b, out_a, out_b,
                    sem_in0, sem_in1, sem_out0, sem_out1):
        src_bufs = (src_a, src_b)
        out_bufs = (out_a, out_b)
        sems_in = (sem_in0, sem_in1)
        sems_out = (sem_out0, sem_out1)
        wid = lax.axis_index("s") * n_cores + lax.axis_index("c")

        def win_start(c):
            return c * chunk

        h_in = [None, None]
        h_in[0] = pltpu.async_copy(
            tgt_hbm.at[pl.ds(win_start(wid), chunk + B)],
            src_a.at[pl.ds(0, chunk + B)], sems_in[0])

        pltpu.sync_copy(acc_hbm, acc_v)
        pltpu.sync_copy(fil_hbm, fil_v)
        pltpu.sync_copy(tbl_hbm, tbl_v.at[pl.ds(0, n_chunks)])
        my_chunks = [wid] + [
            _fetch(tbl_v, wid * cpw + i) for i in range(1, cpw)]

        def pfx(i, carry):
            ca, cf = carry
            iota = lax.iota(jnp.int32, L)
            va = acc_v[pl.ds(i * L, L)]
            vf = fil_v[pl.ds(i * L, L)]
            csa = _cumsum16(tmp_v, va, iota) + ca
            csf = _cumsum16(tmp_v, vf, iota) + cf
            gidx = iota + i * L
            ns = csf - vf
            cs_v[pl.ds(i * L, L)] = csf
            ns_v[pl.ds(i * L, L)] = ns
            sh_v[pl.ds(i * L, L)] = csa - va + gidx - ns
            return csa[L - 1], csf[L - 1]

        lax.fori_loop(0, nvec, pfx, (jnp.int32(0), jnp.int32(0)))

        h_out = [None, None, None, None]
        for i, c in enumerate(my_chunks):
            buf = i % 2
            if i + 1 < cpw:
                h_in[1 - buf] = pltpu.async_copy(
                    tgt_hbm.at[pl.ds(win_start(my_chunks[i + 1]), chunk + B)],
                    src_bufs[1 - buf].at[pl.ds(0, chunk + B)], sems_in[1 - buf])
            h_in[buf].wait()
            if i >= 2:
                h_out[i - 2].wait()

            o0 = c * chunk
            end = o0 + chunk
            s_lo = _search_right(cs_v, o0, log2b)
            s_hi = _search_right(ns_v, end - 1, log2b)

            @plsc.parallel_loop(s_lo, s_hi, jnp.int32(1))
            def seg_body(s, buf=buf, o0=o0, end=end):
                ns = _fetch(ns_v, s)
                cf = _fetch(cs_v, s)
                sh = _fetch(sh_v, s)
                run_lo = jnp.maximum(ns, o0)
                run_hi = jnp.minimum(cf, end)
                lo_out = run_lo - o0
                lo_src = lo_out + sh
                ln = run_hi - run_lo
                full = ln // L * L

                @plsc.parallel_loop(jnp.int32(0), full, jnp.int32(L),
                                    unroll=UNROLL)
                def cp(b0):
                    out_bufs[buf][pl.ds(lo_out + b0, L)] = (
                        src_bufs[buf][pl.ds(lo_src + b0, L)])

                iota = lax.iota(jnp.int32, L)
                rem = ln - full
                v = src_bufs[buf][pl.ds(lo_src + full, L)]
                plsc.store_scatter(
                    out_bufs[buf], [iota + (lo_out + full)], v,
                    mask=iota < rem)
            h_out[i] = pltpu.async_copy(
                out_bufs[buf].at[pl.ds(0, chunk)],
                out_hbm.at[pl.ds(o0, chunk)], sems_out[buf])

        h_out[cpw - 2].wait()
        h_out[cpw - 1].wait()

    return ragged_copy, tuple(int(x) for x in table_np)


def kernel(tgt_cache_loc, accept_length, accept_length_filter):
    B = accept_length.shape[0]
    fn, table = _build(B, tgt_cache_loc.shape[0])
    return fn(jnp.asarray(tgt_cache_loc, jnp.float32),
              jnp.asarray(accept_length, jnp.int32),
              jnp.asarray(accept_length_filter, jnp.int32),
              jnp.asarray(table, jnp.int32))


# speedup vs baseline: 1.1198x; 1.1198x over previous
import functools

import jax
import jax.numpy as jnp
import numpy as np
from jax import lax
from jax.experimental import pallas as pl
from jax.experimental.pallas import tpu as pltpu
from jax.experimental.pallas import tpu_sc as plsc

L = 16
UNROLL = 8
STEP = L * UNROLL


_GATHER_DNUMS = lax.GatherDimensionNumbers(
    offset_dims=(), collapsed_slice_dims=(0,), start_index_map=(0,))


def _lane_gather(v, idx):
    return lax.gather(v, idx[:, None], _GATHER_DNUMS, slice_sizes=(1,),
                      mode=lax.GatherScatterMode.PROMISE_IN_BOUNDS)


def _cumsum16(v, iota):
    for k in (1, 2, 4, 8):
        sh = _lane_gather(v, jnp.maximum(iota - k, 0))
        v = v + jnp.where(iota >= k, sh, 0)
    return v


def _fetch(ref, i):
    return ref[pl.ds(i, L)][0]


def _search_right(ref, x):
    lo = jnp.int32(0)
    iota = lax.iota(jnp.int32, L)
    for s in (128, 8, 1):
        q = lo + iota * s + (s - 1)
        v = plsc.load_gather(ref, [q])
        cnt = plsc.all_reduce_population_count(v <= x)[0]
        lo = lo + cnt * s
    return lo


@functools.lru_cache(maxsize=None)
def _build(B, tgt_len):
    out_size = B * (B - 1) // 2
    n_chunks = 128
    assert out_size % n_chunks == 0
    chunk = out_size // n_chunks
    assert chunk % 8 == 0 and tgt_len == out_size + B
    log2b = B.bit_length() - 1
    assert 2 ** log2b == B and B == 128 * L
    nvec = B // L

    mesh = plsc.VectorSubcoreMesh(core_axis_name="c", subcore_axis_name="s")
    n_cores = 2
    n_workers = 32
    cpw = n_chunks // n_workers

    fil_s = np.arange(B, dtype=np.int64)
    cs_s = np.cumsum(fil_s)
    ns_s = cs_s - fil_s
    segcnt = []
    for c in range(n_chunks):
        o0_s = c * chunk
        s_lo_s = int(np.searchsorted(cs_s, o0_s, side="right"))
        s_hi_s = int(np.searchsorted(ns_s, o0_s + chunk - 1, side="right"))
        segcnt.append(s_hi_s - s_lo_s)
    bins = [[segcnt[w], [w]] for w in range(n_workers)]
    for c in sorted(range(n_workers, n_chunks), key=lambda c: -segcnt[c]):
        b = min((b for b in bins if len(b[1]) < cpw), key=lambda b: b[0])
        b[0] += segcnt[c]
        b[1].append(c)
    table_np = np.array([b[1] for b in bins], dtype=np.int32).reshape(-1)

    @functools.partial(
        pl.kernel,
        out_type=jax.ShapeDtypeStruct((out_size,), jnp.float32),
        mesh=mesh,
        compiler_params=pltpu.CompilerParams(needs_layout_passes=False),
        scratch_types=[
            pltpu.VMEM((B,), jnp.int32),
            pltpu.VMEM((B,), jnp.int32),
            pltpu.VMEM((B + 144,), jnp.int32),
            pltpu.VMEM((B + 144,), jnp.int32),
            pltpu.VMEM((B + L,), jnp.int32),
            pltpu.VMEM((n_chunks + L,), jnp.int32),
            pltpu.VMEM((chunk + B + STEP,), jnp.float32),
            pltpu.VMEM((chunk + B + STEP,), jnp.float32),
            pltpu.VMEM((chunk + STEP,), jnp.float32),
            pltpu.VMEM((chunk + STEP,), jnp.float32),
            pltpu.SemaphoreType.DMA,
            pltpu.SemaphoreType.DMA,
            pltpu.SemaphoreType.DMA,
            pltpu.SemaphoreType.DMA,
        ],
    )
    def ragged_copy(tgt_hbm, acc_hbm, fil_hbm, tbl_hbm, out_hbm,
                    acc_v, fil_v, cs_v, ns_v, sh_v, tbl_v,
                    src_a, src_b, out_a, out_b,
                    sem_in0, sem_in1, sem_out0, sem_out1):
        src_bufs = (src_a, src_b)
        out_bufs = (out_a, out_b)
        sems_in = (sem_in0, sem_in1)
        sems_out = (sem_out0, sem_out1)
        wid = lax.axis_index("s") * n_cores + lax.axis_index("c")

        def win_start(c):
            return c * chunk

        h_in = [None, None]
        h_in[0] = pltpu.async_copy(
            tgt_hbm.at[pl.ds(win_start(wid), chunk + B)],
            src_a.at[pl.ds(0, chunk + B)], sems_in[0])

        pltpu.sync_copy(acc_hbm, acc_v)
        pltpu.sync_copy(fil_hbm, fil_v)
        pltpu.sync_copy(tbl_hbm, tbl_v.at[pl.ds(0, n_chunks)])
        my_chunks = [wid] + [
            _fetch(tbl_v, wid * cpw + i) for i in range(1, cpw)]

        def pfx(i, carry):
            ca, cf = carry
            iota = lax.iota(jnp.int32, L)
            va = acc_v[pl.ds(i * L, L)]
            vf = fil_v[pl.ds(i * L, L)]
            csa = _cumsum16(va, iota) + ca
            csf = _cumsum16(vf, iota) + cf
            gidx = iota + i * L
            ns = csf - vf
            cs_v[pl.ds(i * L, L)] = csf
            ns_v[pl.ds(i * L, L)] = ns
            sh_v[pl.ds(i * L, L)] = csa - va + gidx - ns
            return csa[L - 1], csf[L - 1]

        lax.fori_loop(0, nvec, pfx, (jnp.int32(0), jnp.int32(0)))

        big = jnp.full((L,), jnp.iinfo(jnp.int32).max, jnp.int32)
        for p in range(B, B + 144, L):
            cs_v[pl.ds(p, L)] = big
            ns_v[pl.ds(p, L)] = big

        h_out = [None, None, None, None]
        for i, c in enumerate(my_chunks):
            buf = i % 2
            if i + 1 < cpw:
                h_in[1 - buf] = pltpu.async_copy(
                    tgt_hbm.at[pl.ds(win_start(my_chunks[i + 1]), chunk + B)],
                    src_bufs[1 - buf].at[pl.ds(0, chunk + B)], sems_in[1 - buf])
            h_in[buf].wait()
            if i >= 2:
                h_out[i - 2].wait()

            o0 = c * chunk
            end = o0 + chunk
            s_lo = _search_right(cs_v, o0)
            s_hi = _search_right(ns_v, end - 1)

            @plsc.parallel_loop(s_lo, s_hi, jnp.int32(1))
            def seg_body(s, buf=buf, o0=o0, end=end):
                ns = _fetch(ns_v, s)
                cf = _fetch(cs_v, s)
                sh = _fetch(sh_v, s)
                run_lo = jnp.maximum(ns, o0)
                run_hi = jnp.minimum(cf, end)
                lo_out = run_lo - o0
                lo_src = lo_out + sh
                ln = run_hi - run_lo
                full = ln // L * L

                @plsc.parallel_loop(jnp.int32(0), full, jnp.int32(L),
                                    unroll=UNROLL)
                def cp(b0):
                    out_bufs[buf][pl.ds(lo_out + b0, L)] = (
                        src_bufs[buf][pl.ds(lo_src + b0, L)])

                iota = lax.iota(jnp.int32, L)
                rem = ln - full
                v = src_bufs[buf][pl.ds(lo_src + full, L)]
                plsc.store_scatter(
                    out_bufs[buf], [iota + (lo_out + full)], v,
                    mask=iota < rem)
            h_out[i] = pltpu.async_copy(
                out_bufs[buf].at[pl.ds(0, chunk)],
                out_hbm.at[pl.ds(o0, chunk)], sems_out[buf])

        h_out[cpw - 2].wait()
        h_out[cpw - 1].wait()

    return ragged_copy, tuple(int(x) for x in table_np)


def kernel(tgt_cache_loc, accept_length, accept_length_filter):
    B = accept_length.shape[0]
    fn, table = _build(B, tgt_cache_loc.shape[0])
    return fn(jnp.asarray(tgt_cache_loc, jnp.float32),
              jnp.asarray(accept_length, jnp.int32),
              jnp.asarray(accept_length_filter, jnp.int32),
              jnp.asarray(table, jnp.int32))
